# Initial kernel scaffold; baseline (speedup 1.0000x reference)
import functools

import jax
import jax.numpy as jnp
import numpy as np
from jax import lax
from jax.experimental import pallas as pl
from jax.experimental.pallas import tpu as pltpu
from jax.experimental.pallas import tpu_sc as plsc

SDIM = 64
VDIM = 16
V3 = 48
RBF_DIM = 64
CUTOFF = 5.0
NUM_LAYERS = 5
N = 50000
E = 800000

NP = 50176
EP = 819200
EB = 1024
NB = 1024
CH = 128
NHALF = 25000
ACC_R = 25600
STRIPE = ACC_R // 16
ZR = 200
DUMP_IDX = np.int32(1 << 28)

_MU = np.linspace(0.0, CUTOFF, RBF_DIM, dtype=np.float32).reshape(1, RBF_DIM)
_GAMMA = RBF_DIM / CUTOFF
_ATT_COL = SDIM + 2 * VDIM


def _ln_tc(x, g, b):
    mu = jnp.mean(x, axis=-1, keepdims=True)
    xc = x - mu
    var = jnp.mean(xc * xc, axis=-1, keepdims=True)
    return xc / jnp.sqrt(var + 1e-6) * g + b


def _vnorm_tc(vf):
    sq = vf * vf
    n2 = sq[:, :VDIM] + sq[:, VDIM:2 * VDIM] + sq[:, 2 * VDIM:]
    rms = jnp.sqrt(jnp.mean(n2, axis=-1, keepdims=True) + 1e-6)
    return vf / rms


def _tile3(x):
    return jnp.concatenate([x, x, x], axis=1)


def _dot(a, b):
    return jnp.dot(a, b, preferred_element_type=jnp.float32)


def _full_spec(shape):
    nd = len(shape)
    return pl.BlockSpec(shape, lambda i: (0,) * nd)


def _row_spec(width):
    return pl.BlockSpec((NB, width), lambda i: (i, 0))


def _node0_body(s_ref, vf_ref, g_ref, b_ref, ws_ref, wd_ref,
                sbar_ref, vbar_ref, a_ref, bp_ref):
    sb = _ln_tc(s_ref[...], g_ref[...], b_ref[...])
    vb = _vnorm_tc(vf_ref[...])
    sbar_ref[...] = sb
    vbar_ref[...] = vb
    a_ref[...] = _dot(sb, ws_ref[...])
    bp_ref[...] = _dot(sb, wd_ref[...])


def _node0(s_p, vf_p, g, b, ws, wd):
    f32 = jnp.float32
    return pl.pallas_call(
        _node0_body,
        grid=(NP // NB,),
        in_specs=[_row_spec(SDIM), _row_spec(V3),
                  _full_spec((1, SDIM)), _full_spec((1, SDIM)),
                  _full_spec((SDIM, SDIM)), _full_spec((SDIM, SDIM))],
        out_specs=[_row_spec(SDIM), _row_spec(V3), _row_spec(SDIM), _row_spec(SDIM)],
        out_shape=[jax.ShapeDtypeStruct((NP, SDIM), f32),
                   jax.ShapeDtypeStruct((NP, V3), f32),
                   jax.ShapeDtypeStruct((NP, SDIM), f32),
                   jax.ShapeDtypeStruct((NP, SDIM), f32)],
    )(s_p, vf_p, g, b, ws, wd)


def _nodeu_body(sp_ref, vp_ref, ss_ref, vs_ref, cnt_ref,
                wu1_ref, bu1_ref, wu2_ref, bu2_ref,
                g_ref, b_ref, ws_ref, wd_ref, w3_ref,
                sbar_ref, vbar_ref, ts_ref, bp_ref):
    cnt = jnp.maximum(cnt_ref[...], 1.0)
    s_agg = ss_ref[...] / cnt
    v_agg = vs_ref[...] / cnt
    sq = v_agg * v_agg
    n2 = sq[:, :VDIM] + sq[:, VDIM:2 * VDIM] + sq[:, 2 * VDIM:]
    vn = jnp.sqrt(n2 + 1e-6)
    sp = sp_ref[...]
    cat = jnp.concatenate([sp, s_agg, vn], axis=1)
    u = jax.nn.silu(_dot(cat, wu1_ref[...]) + bu1_ref[...])
    u2 = _dot(u, wu2_ref[...]) + bu2_ref[...]
    s_new = sp + u2[:, :SDIM]
    gate = u2[:, SDIM:SDIM + VDIM]
    v_new = vp_ref[...] + _tile3(gate) * v_agg
    sb = _ln_tc(s_new, g_ref[...], b_ref[...])
    vb = _vnorm_tc(v_new)
    sbar_ref[...] = sb
    vbar_ref[...] = vb
    ts_ref[...] = jnp.concatenate(
        [_dot(sb, ws_ref[...]), _dot(vb, w3_ref[...])], axis=1)
    bp_ref[...] = _dot(sb, wd_ref[...])


def _nodeu(sp, vp, ssum, vsum, cnt, wu1, bu1, wu2, bu2, g, b, ws, wd, w3):
    f32 = jnp.float32
    return pl.pallas_call(
        _nodeu_body,
        grid=(NP // NB,),
        in_specs=[_row_spec(SDIM), _row_spec(V3), _row_spec(SDIM), _row_spec(V3),
                  _row_spec(1),
                  _full_spec((2 * SDIM + VDIM, SDIM)), _full_spec((1, SDIM)),
                  _full_spec((SDIM, SDIM + VDIM)), _full_spec((1, SDIM + VDIM)),
                  _full_spec((1, SDIM)), _full_spec((1, SDIM)),
                  _full_spec((SDIM, SDIM)), _full_spec((SDIM, SDIM)),
                  _full_spec((V3, V3))],
        out_specs=[_row_spec(SDIM), _row_spec(V3), _row_spec(SDIM + V3),
                   _row_spec(SDIM)],
        out_shape=[jax.ShapeDtypeStruct((NP, SDIM), f32),
                   jax.ShapeDtypeStruct((NP, V3), f32),
                   jax.ShapeDtypeStruct((NP, SDIM + V3), f32),
                   jax.ShapeDtypeStruct((NP, SDIM), f32)],
    )(sp, vp, ssum, vsum, cnt, wu1, bu1, wu2, bu2, g, b, ws, wd, w3)


def _fin_body(sp_ref, vp_ref, ss_ref, vs_ref, cnt_ref, s_ref, v_ref):
    cnt = jnp.maximum(cnt_ref[...], 1.0)
    s_ref[...] = sp_ref[...] + ss_ref[...] / cnt
    v_ref[...] = vp_ref[...] + vs_ref[...] / cnt


def _final(sp, vp, ssum, vsum, cnt):
    f32 = jnp.float32
    return pl.pallas_call(
        _fin_body,
        grid=(NP // NB,),
        in_specs=[_row_spec(SDIM), _row_spec(V3), _row_spec(SDIM), _row_spec(V3),
                  _row_spec(1)],
        out_specs=[_row_spec(SDIM), _row_spec(V3)],
        out_shape=[jax.ShapeDtypeStruct((NP, SDIM), f32),
                   jax.ShapeDtypeStruct((NP, V3), f32)],
    )(sp, vp, ssum, vsum, cnt)


def _edge_body(gs_ref, gd_ref, dr_ref, mu_ref, w1e_ref, be1_ref, w2_ref,
               be2_ref, sm_ref, vm_ref, *, is_rbf, has_v):
    dr = dr_ref[...]
    d = dr[:, 0:1]
    gs = gs_ref[...]
    ga = gs[:, :SDIM]
    gb = gd_ref[...]
    if is_rbf:
        ef = jnp.exp(-_GAMMA * (d - mu_ref[...]) ** 2)
        pre = ga + gb + _dot(ef, w1e_ref[...])
    else:
        pre = ga + gb + d * w1e_ref[...]
    h = jax.nn.silu(pre + be1_ref[...])
    m = _dot(h, w2_ref[...]) + be2_ref[...]
    att = jax.nn.sigmoid(m[:, _ATT_COL:_ATT_COL + 1])
    if is_rbf:
        env = jnp.where(d < CUTOFF,
                        0.5 * (jnp.cos(jnp.pi / CUTOFF * d) + 1.0), 0.0)
        att = att * env
    sm_ref[...] = m[:, :SDIM] * att
    grv = m[:, SDIM + VDIM:SDIM + 2 * VDIM] * att
    nrow = dr.shape[0]
    r48 = jnp.concatenate([jnp.broadcast_to(dr[:, 1:2], (nrow, VDIM)),
                           jnp.broadcast_to(dr[:, 2:3], (nrow, VDIM)),
                           jnp.broadcast_to(dr[:, 3:4], (nrow, VDIM))], axis=1)
    vm = _tile3(grv) * r48
    if has_v:
        gvv = m[:, SDIM:SDIM + VDIM] * att
        vm = vm + _tile3(gvv) * gs[:, SDIM:]
    vm_ref[...] = vm


def _edge(gsrc, gdst, dr, w1e, be1, w2p, be2p, is_rbf, has_v):
    f32 = jnp.float32
    wsrc = int(gsrc.shape[1])
    body = functools.partial(_edge_body, is_rbf=is_rbf, has_v=has_v)
    return pl.pallas_call(
        body,
        grid=(EP // EB,),
        in_specs=[pl.BlockSpec((EB, wsrc), lambda i: (i, 0)),
                  pl.BlockSpec((EB, SDIM), lambda i: (i, 0)),
                  pl.BlockSpec((EB, 4), lambda i: (i, 0)),
                  _full_spec((1, RBF_DIM)),
                  _full_spec(tuple(w1e.shape)), _full_spec((1, SDIM)),
                  _full_spec((SDIM, 128)), _full_spec((1, 128))],
        out_specs=[pl.BlockSpec((EB, SDIM), lambda i: (i, 0)),
                   pl.BlockSpec((EB, V3), lambda i: (i, 0))],
        out_shape=[jax.ShapeDtypeStruct((EP, SDIM), f32),
                   jax.ShapeDtypeStruct((EP, V3), f32)],
    )(gsrc, gdst, dr, jnp.asarray(_MU), w1e, be1, w2p, be2p)


def _sc_mesh():
    return plsc.VectorSubcoreMesh(core_axis_name="c", subcore_axis_name="s")


@functools.lru_cache(maxsize=None)
def _make_sc_gather(wsrc):
    f32 = jnp.float32

    @functools.partial(
        pl.kernel, mesh=_sc_mesh(),
        out_type=[jax.ShapeDtypeStruct((EP, wsrc), f32),
                  jax.ShapeDtypeStruct((EP, SDIM), f32)],
        scratch_types=[pltpu.VMEM((CH,), jnp.int32),
                       pltpu.VMEM((CH,), jnp.int32),
                       pltpu.VMEM((CH, wsrc), f32),
                       pltpu.VMEM((CH, SDIM), f32),
                       pltpu.SemaphoreType.DMA,
                       pltpu.SemaphoreType.DMA],
    )
    def gath(ts_hbm, td_hbm, src_hbm, dst_hbm, os_hbm, od_hbm,
             si, di, srows, drows, sem_a, sem_b):
        wid = lax.axis_index("s") * 2 + lax.axis_index("c")
        base0 = wid * (EP // 32)

        def body(i, carry):
            base = base0 + i * CH
            pltpu.sync_copy(src_hbm.at[pl.ds(base, CH)], si)
            pltpu.sync_copy(dst_hbm.at[pl.ds(base, CH)], di)
            ca = pltpu.async_copy(ts_hbm.at[si], srows, sem_a)
            cb = pltpu.async_copy(td_hbm.at[di], drows, sem_b)
            ca.wait()
            cb.wait()
            pltpu.sync_copy(srows, os_hbm.at[pl.ds(base, CH)])
            pltpu.sync_copy(drows, od_hbm.at[pl.ds(base, CH)])
            return carry

        lax.fori_loop(0, EP // 32 // CH, body, 0)

    return gath


def _sc_gather(tsrc, tdst, src_idx, dst_idx):
    return _make_sc_gather(int(tsrc.shape[1]))(tsrc, tdst, src_idx, dst_idx)


@functools.lru_cache(maxsize=None)
def _make_sc_scatter(width, with_counts):
    f32 = jnp.float32
    outs = [jax.ShapeDtypeStruct((2, ACC_R, width), f32)]
    scratch = [pltpu.VMEM((ZR, width), f32),
               pltpu.VMEM((CH,), jnp.int32),
               pltpu.VMEM((CH, width), f32),
               pltpu.VMEM_SHARED((ACC_R, width), f32)]
    if with_counts:
        outs.append(jax.ShapeDtypeStruct((2, ACC_R, 16), f32))
        scratch += [pltpu.VMEM((ZR, 16), f32),
                    pltpu.VMEM((CH, 16), f32),
                    pltpu.VMEM_SHARED((ACC_R, 16), f32)]

    @functools.partial(pl.kernel, mesh=_sc_mesh(), out_type=outs,
                       scratch_types=scratch)
    def scat(pay_hbm, dst_hbm, *refs):
        if with_counts:
            out_hbm, cout_hbm, zbuf, idxb, payb, acc, zbufc, onesb, cacc = refs
        else:
            out_hbm, zbuf, idxb, payb, acc = refs
        c = lax.axis_index("c")
        sid = lax.axis_index("s")
        zv = jnp.zeros((16,), f32)

        def zrow(i, carry):
            for j in range(width // 16):
                zbuf[i, pl.ds(j * 16, 16)] = zv
            if with_counts:
                zbufc[i, pl.ds(0, 16)] = zv
            return carry

        lax.fori_loop(0, ZR, zrow, 0)
        if with_counts:
            ov = jnp.ones((16,), f32)

            def orow(i, carry):
                onesb[i, pl.ds(0, 16)] = ov
                return carry

            lax.fori_loop(0, CH, orow, 0)
        row0 = sid * STRIPE
        for jj in range(STRIPE // ZR):
            pltpu.sync_copy(zbuf, acc.at[pl.ds(row0 + jj * ZR, ZR)])
            if with_counts:
                pltpu.sync_copy(zbufc, cacc.at[pl.ds(row0 + jj * ZR, ZR)])
        plsc.subcore_barrier()
        base0 = sid * (EP // 16)
        lo = c * NHALF

        def body(i, carry):
            base = base0 + i * CH
            pltpu.sync_copy(dst_hbm.at[pl.ds(base, CH)], idxb)
            for j in range(CH // 16):
                x = idxb[pl.ds(j * 16, 16)]
                li = x - lo
                ok = (li >= 0) & (li < NHALF)
                idxb[pl.ds(j * 16, 16)] = jnp.where(ok, li, NHALF)
            pltpu.sync_copy(pay_hbm.at[pl.ds(base, CH)], payb)
            pltpu.sync_copy(payb, acc.at[idxb], add=True)
            if with_counts:
                pltpu.sync_copy(onesb, cacc.at[idxb], add=True)
            return carry

        lax.fori_loop(0, EP // 16 // CH, body, 0)
        plsc.subcore_barrier()
        pltpu.sync_copy(acc.at[pl.ds(row0, STRIPE)],
                        out_hbm.at[c, pl.ds(row0, STRIPE)])
        if with_counts:
            pltpu.sync_copy(cacc.at[pl.ds(row0, STRIPE)],
                            cout_hbm.at[c, pl.ds(row0, STRIPE)])

    return scat


def _sc_scatter(pay, dst_idx, with_counts):
    return _make_sc_scatter(int(pay.shape[1]), bool(with_counts))(pay, dst_idx)


def _assemble(acc2):
    full = jnp.concatenate([acc2[0, :NHALF], acc2[1, :NHALF]], axis=0)
    pad = jnp.zeros((NP - N, full.shape[1]), full.dtype)
    return jnp.concatenate([full, pad], axis=0)


def _pad_rows(x, rows, value=0.0):
    pad = jnp.full((rows - x.shape[0],) + x.shape[1:], value, x.dtype)
    return jnp.concatenate([x, pad], axis=0)


def kernel(s, v, p, edge_index_local, d_local, r_local,
           edge_index_global, d_global, r_global, batch, params):
    f32 = jnp.float32
    s_p = _pad_rows(s, NP)
    vf_p = _pad_rows(v.transpose(0, 2, 1).reshape(N, V3), NP)

    def prep_edges(ei, d, r):
        src = _pad_rows(ei[0], EP)
        dst_g = _pad_rows(ei[1], EP)
        dst_s = jnp.concatenate(
            [ei[1], jnp.full((EP - E,), DUMP_IDX, jnp.int32)])
        dr = _pad_rows(jnp.concatenate([d[:, None], r], axis=1), EP)
        return src, dst_g, dst_s, dr

    srcL, dstLg, dstLs, drL = prep_edges(edge_index_local, d_local, r_local)
    srcG, dstGg, dstGs, drG = prep_edges(edge_index_global, d_global, r_global)

    def wts(i):
        lp = params[i]
        we1 = lp['We1']
        ws, wd, w1e = we1[:SDIM], we1[SDIM:2 * SDIM], we1[2 * SDIM:]
        be1 = lp['be1'].reshape(1, SDIM)
        w2p = jnp.zeros((SDIM, 128), f32).at[:, :SDIM + 2 * VDIM + 1].set(lp['We2'])
        be2p = jnp.zeros((1, 128), f32).at[0, :SDIM + 2 * VDIM + 1].set(lp['be2'])
        g = lp['g'].reshape(1, SDIM)
        b = lp['b'].reshape(1, SDIM)
        return ws, wd, w1e, be1, w2p, be2p, g, b

    ws, wd, w1e, be1, w2p, be2p, g, b = wts(0)
    sbar, vbar, a0, bp0 = _node0(s_p, vf_p, g, b, ws, wd)
    gsrc, gdst = _sc_gather(a0, bp0, srcL, dstLg)
    sm, vm = _edge(gsrc, gdst, drL, w1e, be1, w2p, be2p, True, False)
    ssum2 = _sc_scatter(sm, dstLs, False)
    vsum2, cnt2 = _sc_scatter(vm, dstLs, True)
    ssum, vsum = _assemble(ssum2), _assemble(vsum2)
    cntL = jnp.maximum(_assemble(cnt2)[:, :1], 1.0)
    cnt = cntL

    for i in range(1, NUM_LAYERS):
        lp_prev = params[i - 1]
        ws, wd, w1e, be1, w2p, be2p, g, b = wts(i)
        w3 = jnp.kron(jnp.eye(3, dtype=f32), params[i]['Wv'])
        sbar, vbar, tsrc, bp = _nodeu(
            sbar, vbar, ssum, vsum, cnt,
            lp_prev['Wu1'], lp_prev['bu1'].reshape(1, SDIM),
            lp_prev['Wu2'], lp_prev['bu2'].reshape(1, SDIM + VDIM),
            g, b, ws, wd, w3)
        is_rbf = (i != NUM_LAYERS - 2)
        if is_rbf:
            src, dstg, dsts, dr = srcL, dstLg, dstLs, drL
        else:
            src, dstg, dsts, dr = srcG, dstGg, dstGs, drG
        gsrc, gdst = _sc_gather(tsrc, bp, src, dstg)
        sm, vm = _edge(gsrc, gdst, dr, w1e, be1, w2p, be2p, is_rbf, True)
        ssum2 = _sc_scatter(sm, dsts, False)
        if is_rbf:
            vsum2 = _sc_scatter(vm, dsts, False)
            cnt = cntL
        else:
            vsum2, cnt2 = _sc_scatter(vm, dsts, True)
            cnt = jnp.maximum(_assemble(cnt2)[:, :1], 1.0)
        ssum, vsum = _assemble(ssum2), _assemble(vsum2)

    s_f, v_f = _final(sbar, vbar, ssum, vsum, cnt)
    s_out = s_f[:N]
    v_out = v_f[:N].reshape(N, 3, VDIM).transpose(0, 2, 1)
    return (s_out, v_out, p)


# speedup vs baseline: 13.2977x; 13.2977x over previous
import functools

import jax
import jax.numpy as jnp
import numpy as np
from jax import lax
from jax.experimental import pallas as pl
from jax.experimental.pallas import tpu as pltpu
from jax.experimental.pallas import tpu_sc as plsc

SDIM = 64
VDIM = 16
V3 = 48
RBF_DIM = 64
CUTOFF = 5.0
NUM_LAYERS = 5
N = 50000
E = 800000
W = 128

NP = 50176
EP = 819200
EB = 1024
NB = 1024
CH = 128
RANGE = 12500
ACC_R = 12544
STRIPE = ACC_R // 16
ZR = 98
DUMP_IDX = np.int32(1 << 28)

_MU = np.linspace(0.0, CUTOFF, RBF_DIM, dtype=np.float32).reshape(1, RBF_DIM)
_GAMMA = RBF_DIM / CUTOFF
_ATT_COL = SDIM + 2 * VDIM
_CNT_COL = SDIM + V3


def _ln_tc(x, g, b):
    mu = jnp.mean(x, axis=-1, keepdims=True)
    xc = x - mu
    var = jnp.mean(xc * xc, axis=-1, keepdims=True)
    return xc / jnp.sqrt(var + 1e-6) * g + b


def _vnorm_tc(vf):
    sq = vf * vf
    n2 = sq[:, :VDIM] + sq[:, VDIM:2 * VDIM] + sq[:, 2 * VDIM:]
    rms = jnp.sqrt(jnp.mean(n2, axis=-1, keepdims=True) + 1e-6)
    return vf / rms


def _tile3(x):
    return jnp.concatenate([x, x, x], axis=1)


def _dot(a, b):
    return jnp.dot(a, b, preferred_element_type=jnp.float32)


def _full_spec(shape):
    nd = len(shape)
    return pl.BlockSpec(shape, lambda i: (0,) * nd)


def _row_spec(width):
    return pl.BlockSpec((NB, width), lambda i: (i, 0))


def _zpad(x, width):
    return jnp.concatenate(
        [x, jnp.zeros((x.shape[0], width - x.shape[1]), x.dtype)], axis=1)


def _node0_body(s_ref, vf_ref, g_ref, b_ref, ws_ref, wd_ref,
                sbar_ref, vbar_ref, ts_ref, td_ref):
    sb = _ln_tc(s_ref[...], g_ref[...], b_ref[...])
    vb = _vnorm_tc(vf_ref[...])
    sbar_ref[...] = sb
    vbar_ref[...] = vb
    z = jnp.zeros((sb.shape[0], W - SDIM), jnp.float32)
    ts_ref[...] = jnp.concatenate([_dot(sb, ws_ref[...]), z], axis=1)
    td_ref[...] = jnp.concatenate([_dot(sb, wd_ref[...]), z], axis=1)


def _node0(s_p, vf_p, g, b, ws, wd):
    f32 = jnp.float32
    return pl.pallas_call(
        _node0_body,
        grid=(NP // NB,),
        in_specs=[_row_spec(SDIM), _row_spec(V3),
                  _full_spec((1, SDIM)), _full_spec((1, SDIM)),
                  _full_spec((SDIM, SDIM)), _full_spec((SDIM, SDIM))],
        out_specs=[_row_spec(SDIM), _row_spec(V3), _row_spec(W), _row_spec(W)],
        out_shape=[jax.ShapeDtypeStruct((NP, SDIM), f32),
                   jax.ShapeDtypeStruct((NP, V3), f32),
                   jax.ShapeDtypeStruct((NP, W), f32),
                   jax.ShapeDtypeStruct((NP, W), f32)],
    )(s_p, vf_p, g, b, ws, wd)


def _nodeu_body(sp_ref, vp_ref, agg_ref,
                wu1_ref, bu1_ref, wu2_ref, bu2_ref,
                g_ref, b_ref, ws_ref, wd_ref, w3_ref,
                sbar_ref, vbar_ref, ts_ref, td_ref):
    agg = agg_ref[...]
    cnt = jnp.maximum(agg[:, _CNT_COL:_CNT_COL + 1], 1.0)
    s_agg = agg[:, :SDIM] / cnt
    v_agg = agg[:, SDIM:SDIM + V3] / cnt
    sq = v_agg * v_agg
    n2 = sq[:, :VDIM] + sq[:, VDIM:2 * VDIM] + sq[:, 2 * VDIM:]
    vn = jnp.sqrt(n2 + 1e-6)
    sp = sp_ref[...]
    cat = jnp.concatenate([sp, s_agg, vn], axis=1)
    u = jax.nn.silu(_dot(cat, wu1_ref[...]) + bu1_ref[...])
    u2 = _dot(u, wu2_ref[...]) + bu2_ref[...]
    s_new = sp + u2[:, :SDIM]
    gate = u2[:, SDIM:SDIM + VDIM]
    v_new = vp_ref[...] + _tile3(gate) * v_agg
    sb = _ln_tc(s_new, g_ref[...], b_ref[...])
    vb = _vnorm_tc(v_new)
    sbar_ref[...] = sb
    vbar_ref[...] = vb
    z = jnp.zeros((sb.shape[0], W - SDIM - V3), jnp.float32)
    ts_ref[...] = jnp.concatenate(
        [_dot(sb, ws_ref[...]), _dot(vb, w3_ref[...]), z], axis=1)
    z2 = jnp.zeros((sb.shape[0], W - SDIM), jnp.float32)
    td_ref[...] = jnp.concatenate([_dot(sb, wd_ref[...]), z2], axis=1)


def _nodeu(sp, vp, agg, wu1, bu1, wu2, bu2, g, b, ws, wd, w3):
    f32 = jnp.float32
    return pl.pallas_call(
        _nodeu_body,
        grid=(NP // NB,),
        in_specs=[_row_spec(SDIM), _row_spec(V3), _row_spec(W),
                  _full_spec((2 * SDIM + VDIM, SDIM)), _full_spec((1, SDIM)),
                  _full_spec((SDIM, SDIM + VDIM)), _full_spec((1, SDIM + VDIM)),
                  _full_spec((1, SDIM)), _full_spec((1, SDIM)),
                  _full_spec((SDIM, SDIM)), _full_spec((SDIM, SDIM)),
                  _full_spec((V3, V3))],
        out_specs=[_row_spec(SDIM), _row_spec(V3), _row_spec(W), _row_spec(W)],
        out_shape=[jax.ShapeDtypeStruct((NP, SDIM), f32),
                   jax.ShapeDtypeStruct((NP, V3), f32),
                   jax.ShapeDtypeStruct((NP, W), f32),
                   jax.ShapeDtypeStruct((NP, W), f32)],
    )(sp, vp, agg, wu1, bu1, wu2, bu2, g, b, ws, wd, w3)


def _fin_body(sp_ref, vp_ref, agg_ref, s_ref, v_ref):
    agg = agg_ref[...]
    cnt = jnp.maximum(agg[:, _CNT_COL:_CNT_COL + 1], 1.0)
    s_ref[...] = sp_ref[...] + agg[:, :SDIM] / cnt
    v_ref[...] = vp_ref[...] + agg[:, SDIM:SDIM + V3] / cnt


def _final(sp, vp, agg):
    f32 = jnp.float32
    return pl.pallas_call(
        _fin_body,
        grid=(NP // NB,),
        in_specs=[_row_spec(SDIM), _row_spec(V3), _row_spec(W)],
        out_specs=[_row_spec(SDIM), _row_spec(V3)],
        out_shape=[jax.ShapeDtypeStruct((NP, SDIM), f32),
                   jax.ShapeDtypeStruct((NP, V3), f32)],
    )(sp, vp, agg)


def _edge_body(gs_ref, gd_ref, drt_ref, mu_ref, w1e_ref, be1_ref, w2_ref,
               be2_ref, msg_ref, *, is_rbf, has_v):
    t = jnp.swapaxes(drt_ref[...], 0, 1)
    d = t[:, 0:1]
    gs = gs_ref[...]
    ga = gs[:, :SDIM]
    gb = gd_ref[:, :SDIM]
    if is_rbf:
        ef = jnp.exp(-_GAMMA * (d - mu_ref[...]) ** 2)
        pre = ga + gb + _dot(ef, w1e_ref[...])
    else:
        pre = ga + gb + d * w1e_ref[...]
    h = jax.nn.silu(pre + be1_ref[...])
    m = _dot(h, w2_ref[...]) + be2_ref[...]
    att = jax.nn.sigmoid(m[:, _ATT_COL:_ATT_COL + 1])
    if is_rbf:
        env = jnp.where(d < CUTOFF,
                        0.5 * (jnp.cos(jnp.pi / CUTOFF * d) + 1.0), 0.0)
        att = att * env
    sm = m[:, :SDIM] * att
    grv = m[:, SDIM + VDIM:SDIM + 2 * VDIM] * att
    nrow = t.shape[0]
    r48 = jnp.concatenate([jnp.broadcast_to(t[:, 1:2], (nrow, VDIM)),
                           jnp.broadcast_to(t[:, 2:3], (nrow, VDIM)),
                           jnp.broadcast_to(t[:, 3:4], (nrow, VDIM))], axis=1)
    vm = _tile3(grv) * r48
    if has_v:
        gvv = m[:, SDIM:SDIM + VDIM] * att
        vm = vm + _tile3(gvv) * gs[:, SDIM:SDIM + V3]
    one = jnp.ones((nrow, 1), jnp.float32)
    z = jnp.zeros((nrow, W - _CNT_COL - 1), jnp.float32)
    msg_ref[...] = jnp.concatenate([sm, vm, one, z], axis=1)


def _edge(gsrc, gdst, drt, w1e, be1, w2p, be2p, is_rbf, has_v):
    f32 = jnp.float32
    body = functools.partial(_edge_body, is_rbf=is_rbf, has_v=has_v)
    return pl.pallas_call(
        body,
        grid=(EP // EB,),
        in_specs=[pl.BlockSpec((EB, W), lambda i: (i, 0)),
                  pl.BlockSpec((EB, W), lambda i: (i, 0)),
                  pl.BlockSpec((8, EB), lambda i: (0, i)),
                  _full_spec((1, RBF_DIM)),
                  _full_spec(tuple(w1e.shape)), _full_spec((1, SDIM)),
                  _full_spec((SDIM, 128)), _full_spec((1, 128))],
        out_specs=[pl.BlockSpec((EB, W), lambda i: (i, 0))],
        out_shape=[jax.ShapeDtypeStruct((EP, W), f32)],
    )(gsrc, gdst, drt, jnp.asarray(_MU), w1e, be1, w2p, be2p)[0]


def _sc_mesh():
    return plsc.VectorSubcoreMesh(core_axis_name="c", subcore_axis_name="s")


@functools.lru_cache(maxsize=None)
def _make_sc_gather():
    f32 = jnp.float32

    @functools.partial(
        pl.kernel, mesh=_sc_mesh(),
        out_type=[jax.ShapeDtypeStruct((EP, W), f32),
                  jax.ShapeDtypeStruct((EP, W), f32)],
        scratch_types=[pltpu.VMEM((CH,), jnp.int32),
                       pltpu.VMEM((CH,), jnp.int32),
                       pltpu.VMEM((CH, W), f32),
                       pltpu.VMEM((CH, W), f32),
                       pltpu.SemaphoreType.DMA,
                       pltpu.SemaphoreType.DMA],
    )
    def gath(ts_hbm, td_hbm, src_hbm, dst_hbm, os_hbm, od_hbm,
             si, di, srows, drows, sem_a, sem_b):
        wid = lax.axis_index("s") * 2 + lax.axis_index("c")
        base0 = wid * (EP // 32)

        def body(i, carry):
            base = base0 + i * CH
            pltpu.sync_copy(src_hbm.at[pl.ds(base, CH)], si)
            pltpu.sync_copy(dst_hbm.at[pl.ds(base, CH)], di)
            ca = pltpu.async_copy(ts_hbm.at[si], srows, sem_a)
            cb = pltpu.async_copy(td_hbm.at[di], drows, sem_b)
            ca.wait()
            cb.wait()
            pltpu.sync_copy(srows, os_hbm.at[pl.ds(base, CH)])
            pltpu.sync_copy(drows, od_hbm.at[pl.ds(base, CH)])
            return carry

        lax.fori_loop(0, EP // 32 // CH, body, 0)

    return gath


def _sc_gather(tsrc, tdst, src_idx, dst_idx):
    res = _make_sc_gather()(tsrc, tdst, src_idx, dst_idx)
    return res[0], res[1]


@functools.lru_cache(maxsize=None)
def _make_sc_scatter():
    f32 = jnp.float32

    @functools.partial(
        pl.kernel, mesh=_sc_mesh(),
        out_type=[jax.ShapeDtypeStruct((4, ACC_R, W), f32)],
        scratch_types=[pltpu.VMEM((ZR, W), f32),
                       pltpu.VMEM((CH,), jnp.int32),
                       pltpu.VMEM((CH, W), f32),
                       pltpu.VMEM_SHARED((ACC_R, W), f32)],
    )
    def scat(pay_hbm, dst_hbm, out_hbm, zbuf, idxb, payb, acc):
        c = lax.axis_index("c")
        sid = lax.axis_index("s")
        zv = jnp.zeros((16,), f32)

        def zrow(i, carry):
            for j in range(W // 16):
                zbuf[i, pl.ds(j * 16, 16)] = zv
            return carry

        lax.fori_loop(0, ZR, zrow, 0)
        row0 = sid * STRIPE
        base0 = sid * (EP // 16)
        for qi in range(2):
            q = 2 * c + qi
            lo = q * RANGE
            for jj in range(STRIPE // ZR):
                pltpu.sync_copy(zbuf, acc.at[pl.ds(row0 + jj * ZR, ZR)])
            plsc.subcore_barrier()

            def body(i, carry):
                base = base0 + i * CH
                pltpu.sync_copy(dst_hbm.at[pl.ds(base, CH)], idxb)
                for j in range(CH // 16):
                    x = idxb[pl.ds(j * 16, 16)]
                    li = x - lo
                    ok = (li >= 0) & (li < RANGE)
                    idxb[pl.ds(j * 16, 16)] = jnp.where(ok, li, RANGE)
                pltpu.sync_copy(pay_hbm.at[pl.ds(base, CH)], payb)
                pltpu.sync_copy(payb, acc.at[idxb], add=True)
                return carry

            lax.fori_loop(0, EP // 16 // CH, body, 0)
            plsc.subcore_barrier()
            pltpu.sync_copy(acc.at[pl.ds(row0, STRIPE)],
                            out_hbm.at[q, pl.ds(row0, STRIPE)])

    return scat


def _sc_scatter(pay, dst_idx):
    return _make_sc_scatter()(pay, dst_idx)[0]


def _assemble(acc4):
    full = jnp.concatenate([acc4[q, :RANGE] for q in range(4)], axis=0)
    pad = jnp.zeros((NP - N, W), full.dtype)
    return jnp.concatenate([full[:N], pad], axis=0)


def _pad_rows(x, rows, value=0.0):
    pad = jnp.full((rows - x.shape[0],) + x.shape[1:], value, x.dtype)
    return jnp.concatenate([x, pad], axis=0)


def kernel(s, v, p, edge_index_local, d_local, r_local,
           edge_index_global, d_global, r_global, batch, params):
    f32 = jnp.float32
    s_p = _pad_rows(s, NP)
    vf_p = _pad_rows(v.transpose(0, 2, 1).reshape(N, V3), NP)

    def prep_edges(ei, d, r):
        src = _pad_rows(ei[0], EP)
        dst_g = _pad_rows(ei[1], EP)
        dst_s = jnp.concatenate(
            [ei[1], jnp.full((EP - E,), DUMP_IDX, jnp.int32)])
        drt = jnp.concatenate(
            [d[None, :], r.T, jnp.zeros((4, E), f32)], axis=0)
        drt = jnp.concatenate([drt, jnp.zeros((8, EP - E), f32)], axis=1)
        return src, dst_g, dst_s, drt

    srcL, dstLg, dstLs, drL = prep_edges(edge_index_local, d_local, r_local)
    srcG, dstGg, dstGs, drG = prep_edges(edge_index_global, d_global, r_global)

    def wts(i):
        lp = params[i]
        we1 = lp['We1']
        ws, wd, w1e = we1[:SDIM], we1[SDIM:2 * SDIM], we1[2 * SDIM:]
        be1 = lp['be1'].reshape(1, SDIM)
        w2p = jnp.zeros((SDIM, 128), f32).at[:, :SDIM + 2 * VDIM + 1].set(lp['We2'])
        be2p = jnp.zeros((1, 128), f32).at[0, :SDIM + 2 * VDIM + 1].set(lp['be2'])
        g = lp['g'].reshape(1, SDIM)
        b = lp['b'].reshape(1, SDIM)
        return ws, wd, w1e, be1, w2p, be2p, g, b

    ws, wd, w1e, be1, w2p, be2p, g, b = wts(0)
    sbar, vbar, tsrc, tdst = _node0(s_p, vf_p, g, b, ws, wd)
    gsrc, gdst = _sc_gather(tsrc, tdst, srcL, dstLg)
    msg = _edge(gsrc, gdst, drL, w1e, be1, w2p, be2p, True, False)
    agg = _assemble(_sc_scatter(msg, dstLs))

    for i in range(1, NUM_LAYERS):
        lp_prev = params[i - 1]
        ws, wd, w1e, be1, w2p, be2p, g, b = wts(i)
        w3 = jnp.kron(jnp.eye(3, dtype=f32), params[i]['Wv'])
        sbar, vbar, tsrc, tdst = _nodeu(
            sbar, vbar, agg,
            lp_prev['Wu1'], lp_prev['bu1'].reshape(1, SDIM),
            lp_prev['Wu2'], lp_prev['bu2'].reshape(1, SDIM + VDIM),
            g, b, ws, wd, w3)
        is_rbf = (i != NUM_LAYERS - 2)
        if is_rbf:
            src, dstg, dsts, drt = srcL, dstLg, dstLs, drL
        else:
            src, dstg, dsts, drt = srcG, dstGg, dstGs, drG
        gsrc, gdst = _sc_gather(tsrc, tdst, src, dstg)
        msg = _edge(gsrc, gdst, drt, w1e, be1, w2p, be2p, is_rbf, True)
        agg = _assemble(_sc_scatter(msg, dsts))

    s_f, v_f = _final(sbar, vbar, agg)
    s_out = s_f[:N]
    v_out = v_f[:N].reshape(N, 3, VDIM).transpose(0, 2, 1)
    return (s_out, v_out, p)


# speedup vs baseline: 14.2596x; 1.0723x over previous
import functools

import jax
import jax.numpy as jnp
import numpy as np
from jax import lax
from jax.experimental import pallas as pl
from jax.experimental.pallas import tpu as pltpu
from jax.experimental.pallas import tpu_sc as plsc

SDIM = 64
VDIM = 16
V3 = 48
RBF_DIM = 64
CUTOFF = 5.0
NUM_LAYERS = 5
N = 50000
E = 800000
W = 128

NP = 50176
EP = 819200
EB = 1024
NB = 1024
CH = 128
RANGE = 12500
ACC_R = 12544
STRIPE = ACC_R // 16
ZR = 98
DUMP_IDX = np.int32(1 << 28)

_MU = np.linspace(0.0, CUTOFF, RBF_DIM, dtype=np.float32).reshape(1, RBF_DIM)
_GAMMA = RBF_DIM / CUTOFF
_ATT_COL = SDIM + 2 * VDIM
_CNT_COL = SDIM + V3


def _ln_tc(x, g, b):
    mu = jnp.mean(x, axis=-1, keepdims=True)
    xc = x - mu
    var = jnp.mean(xc * xc, axis=-1, keepdims=True)
    return xc / jnp.sqrt(var + 1e-6) * g + b


def _vnorm_tc(vf):
    sq = vf * vf
    n2 = sq[:, :VDIM] + sq[:, VDIM:2 * VDIM] + sq[:, 2 * VDIM:]
    rms = jnp.sqrt(jnp.mean(n2, axis=-1, keepdims=True) + 1e-6)
    return vf / rms


def _tile3(x):
    return jnp.concatenate([x, x, x], axis=1)


def _dot(a, b):
    return jnp.dot(a, b, preferred_element_type=jnp.float32)


def _full_spec(shape):
    nd = len(shape)
    return pl.BlockSpec(shape, lambda i: (0,) * nd)


def _row_spec(width):
    return pl.BlockSpec((NB, width), lambda i: (i, 0))


def _zpad(x, width):
    return jnp.concatenate(
        [x, jnp.zeros((x.shape[0], width - x.shape[1]), x.dtype)], axis=1)


def _node0_body(s_ref, vf_ref, g_ref, b_ref, ws_ref, wd_ref,
                sbar_ref, vbar_ref, ts_ref, td_ref):
    sb = _ln_tc(s_ref[...], g_ref[...], b_ref[...])
    vb = _vnorm_tc(vf_ref[...])
    sbar_ref[...] = sb
    vbar_ref[...] = vb
    z = jnp.zeros((sb.shape[0], W - SDIM), jnp.float32)
    ts_ref[...] = jnp.concatenate(
        [_dot(sb, ws_ref[...]), z], axis=1)
    td_ref[...] = jnp.concatenate([_dot(sb, wd_ref[...]), z], axis=1)


def _node0(s_p, vf_p, g, b, ws, wd):
    f32 = jnp.float32
    return pl.pallas_call(
        _node0_body,
        grid=(NP // NB,),
        in_specs=[_row_spec(SDIM), _row_spec(V3),
                  _full_spec((1, SDIM)), _full_spec((1, SDIM)),
                  _full_spec((SDIM, SDIM)), _full_spec((SDIM, SDIM))],
        out_specs=[_row_spec(SDIM), _row_spec(V3), _row_spec(W), _row_spec(W)],
        out_shape=[jax.ShapeDtypeStruct((NP, SDIM), f32),
                   jax.ShapeDtypeStruct((NP, V3), f32),
                   jax.ShapeDtypeStruct((NP, W), f32),
                   jax.ShapeDtypeStruct((NP, W), f32)],
    )(s_p, vf_p, g, b, ws, wd)


def _nodeu_body(sp_ref, vp_ref, agg_ref,
                wu1_ref, bu1_ref, wu2_ref, bu2_ref,
                g_ref, b_ref, ws_ref, wd_ref, w3_ref,
                sbar_ref, vbar_ref, ts_ref, td_ref):
    agg = agg_ref[...]
    cnt = jnp.maximum(agg[:, _CNT_COL:_CNT_COL + 1], 1.0)
    s_agg = agg[:, :SDIM] / cnt
    v_agg = agg[:, SDIM:SDIM + V3] / cnt
    sq = v_agg * v_agg
    n2 = sq[:, :VDIM] + sq[:, VDIM:2 * VDIM] + sq[:, 2 * VDIM:]
    vn = jnp.sqrt(n2 + 1e-6)
    sp = sp_ref[...]
    cat = jnp.concatenate([sp, s_agg, vn], axis=1)
    u = jax.nn.silu(_dot(cat, wu1_ref[...]) + bu1_ref[...])
    u2 = _dot(u, wu2_ref[...]) + bu2_ref[...]
    s_new = sp + u2[:, :SDIM]
    gate = u2[:, SDIM:SDIM + VDIM]
    v_new = vp_ref[...] + _tile3(gate) * v_agg
    sb = _ln_tc(s_new, g_ref[...], b_ref[...])
    vb = _vnorm_tc(v_new)
    sbar_ref[...] = sb
    vbar_ref[...] = vb
    z = jnp.zeros((sb.shape[0], W - SDIM - V3), jnp.float32)
    ts_ref[...] = jnp.concatenate(
        [_dot(sb, ws_ref[...]), _dot(vb, w3_ref[...]), z], axis=1)
    z2 = jnp.zeros((sb.shape[0], W - SDIM), jnp.float32)
    td_ref[...] = jnp.concatenate([_dot(sb, wd_ref[...]), z2], axis=1)


def _nodeu(sp, vp, agg, wu1, bu1, wu2, bu2, g, b, ws, wd, w3):
    f32 = jnp.float32
    return pl.pallas_call(
        _nodeu_body,
        grid=(NP // NB,),
        in_specs=[_row_spec(SDIM), _row_spec(V3), _row_spec(W),
                  _full_spec((2 * SDIM + VDIM, SDIM)), _full_spec((1, SDIM)),
                  _full_spec((SDIM, SDIM + VDIM)), _full_spec((1, SDIM + VDIM)),
                  _full_spec((1, SDIM)), _full_spec((1, SDIM)),
                  _full_spec((SDIM, SDIM)), _full_spec((SDIM, SDIM)),
                  _full_spec((V3, V3))],
        out_specs=[_row_spec(SDIM), _row_spec(V3), _row_spec(W), _row_spec(W)],
        out_shape=[jax.ShapeDtypeStruct((NP, SDIM), f32),
                   jax.ShapeDtypeStruct((NP, V3), f32),
                   jax.ShapeDtypeStruct((NP, W), f32),
                   jax.ShapeDtypeStruct((NP, W), f32)],
    )(sp, vp, agg, wu1, bu1, wu2, bu2, g, b, ws, wd, w3)


def _fin_body(sp_ref, vp_ref, agg_ref, s_ref, v_ref):
    agg = agg_ref[...]
    cnt = jnp.maximum(agg[:, _CNT_COL:_CNT_COL + 1], 1.0)
    s_ref[...] = sp_ref[...] + agg[:, :SDIM] / cnt
    v_ref[...] = vp_ref[...] + agg[:, SDIM:SDIM + V3] / cnt


def _final(sp, vp, agg):
    f32 = jnp.float32
    return pl.pallas_call(
        _fin_body,
        grid=(NP // NB,),
        in_specs=[_row_spec(SDIM), _row_spec(V3), _row_spec(W)],
        out_specs=[_row_spec(SDIM), _row_spec(V3)],
        out_shape=[jax.ShapeDtypeStruct((NP, SDIM), f32),
                   jax.ShapeDtypeStruct((NP, V3), f32)],
    )(sp, vp, agg)


def _edge_body(gs_ref, gd_ref, drt_ref, mu_ref, w1e_ref, be1_ref, w2_ref,
               be2_ref, msg_ref, *, is_rbf, has_v):
    t = jnp.swapaxes(drt_ref[...], 0, 1)
    d = t[:, 0:1]
    gs = gs_ref[...]
    ga = gs[:, :SDIM]
    gb = gd_ref[:, :SDIM]
    if is_rbf:
        ef = jnp.exp(-_GAMMA * (d - mu_ref[...]) ** 2)
        pre = ga + gb + _dot(ef, w1e_ref[...])
    else:
        pre = ga + gb + d * w1e_ref[...]
    h = jax.nn.silu(pre + be1_ref[...])
    m = _dot(h, w2_ref[...]) + be2_ref[...]
    att = jax.nn.sigmoid(m[:, _ATT_COL:_ATT_COL + 1])
    if is_rbf:
        env = jnp.where(d < CUTOFF,
                        0.5 * (jnp.cos(jnp.pi / CUTOFF * d) + 1.0), 0.0)
        att = att * env
    sm = m[:, :SDIM] * att
    grv = m[:, SDIM + VDIM:SDIM + 2 * VDIM] * att
    nrow = t.shape[0]
    r48 = jnp.concatenate([jnp.broadcast_to(t[:, 1:2], (nrow, VDIM)),
                           jnp.broadcast_to(t[:, 2:3], (nrow, VDIM)),
                           jnp.broadcast_to(t[:, 3:4], (nrow, VDIM))], axis=1)
    vm = _tile3(grv) * r48
    if has_v:
        gvv = m[:, SDIM:SDIM + VDIM] * att
        vm = vm + _tile3(gvv) * gs[:, SDIM:SDIM + V3]
    one = jnp.ones((nrow, 1), jnp.float32)
    z = jnp.zeros((nrow, W - _CNT_COL - 1), jnp.float32)
    msg_ref[...] = jnp.concatenate([sm, vm, one, z], axis=1)


def _edge(gsrc, gdst, drt, w1e, be1, w2p, be2p, is_rbf, has_v):
    f32 = jnp.float32
    body = functools.partial(_edge_body, is_rbf=is_rbf, has_v=has_v)
    return pl.pallas_call(
        body,
        grid=(EP // EB,),
        in_specs=[pl.BlockSpec((EB, W), lambda i: (i, 0)),
                  pl.BlockSpec((EB, W), lambda i: (i, 0)),
                  pl.BlockSpec((8, EB), lambda i: (0, i)),
                  _full_spec((1, RBF_DIM)),
                  _full_spec(tuple(w1e.shape)), _full_spec((1, SDIM)),
                  _full_spec((SDIM, 128)), _full_spec((1, 128))],
        out_specs=[pl.BlockSpec((EB, W), lambda i: (i, 0))],
        out_shape=[jax.ShapeDtypeStruct((EP, W), f32)],
    )(gsrc, gdst, drt, jnp.asarray(_MU), w1e, be1, w2p, be2p)[0]


def _sc_mesh():
    return plsc.VectorSubcoreMesh(core_axis_name="c", subcore_axis_name="s")


GCH = 256
GNC = EP // 16 // GCH


@functools.lru_cache(maxsize=None)
def _make_sc_gather():
    f32 = jnp.float32

    @functools.partial(
        pl.kernel, mesh=_sc_mesh(),
        out_type=[jax.ShapeDtypeStruct((EP, W), f32),
                  jax.ShapeDtypeStruct((EP, W), f32)],
        scratch_types=[pltpu.VMEM((GCH,), jnp.int32),
                       pltpu.VMEM((GCH,), jnp.int32),
                       pltpu.VMEM((GCH, W), f32),
                       pltpu.VMEM((GCH, W), f32),
                       pltpu.SemaphoreType.DMA,
                       pltpu.SemaphoreType.DMA],
    )
    def gath(ts_hbm, td_hbm, src_hbm, dst_hbm, os_hbm, od_hbm,
             ia, ib, bufa, bufb, sem_a, sem_b):
        wid = lax.axis_index("s") * 2 + lax.axis_index("c")
        half = wid // 16
        lane = wid % 16
        base0 = lane * (EP // 16)

        def load_start(cidx, ibuf, rbuf, sem):
            base = base0 + cidx * GCH

            @pl.when(half == 0)
            def _():
                pltpu.sync_copy(src_hbm.at[pl.ds(base, GCH)], ibuf)
                pltpu.async_copy(ts_hbm.at[ibuf.at[pl.ds(0, CH)]],
                                 rbuf.at[pl.ds(0, CH)], sem)
                pltpu.async_copy(ts_hbm.at[ibuf.at[pl.ds(CH, CH)]],
                                 rbuf.at[pl.ds(CH, CH)], sem)

            @pl.when(half == 1)
            def _():
                pltpu.sync_copy(dst_hbm.at[pl.ds(base, GCH)], ibuf)
                pltpu.async_copy(td_hbm.at[ibuf.at[pl.ds(0, CH)]],
                                 rbuf.at[pl.ds(0, CH)], sem)
                pltpu.async_copy(td_hbm.at[ibuf.at[pl.ds(CH, CH)]],
                                 rbuf.at[pl.ds(CH, CH)], sem)

        def drain(rbuf, sem):
            pltpu.make_async_copy(ts_hbm.at[pl.ds(0, CH)],
                                  rbuf.at[pl.ds(0, CH)], sem).wait()
            pltpu.make_async_copy(ts_hbm.at[pl.ds(0, CH)],
                                  rbuf.at[pl.ds(CH, CH)], sem).wait()

        def write(cidx, rbuf):
            base = base0 + cidx * GCH

            @pl.when(half == 0)
            def _():
                pltpu.sync_copy(rbuf, os_hbm.at[pl.ds(base, GCH)])

            @pl.when(half == 1)
            def _():
                pltpu.sync_copy(rbuf, od_hbm.at[pl.ds(base, GCH)])

        load_start(0, ia, bufa, sem_a)

        def body(k, carry):
            load_start(2 * k + 1, ib, bufb, sem_b)
            drain(bufa, sem_a)
            write(2 * k, bufa)

            @pl.when(k < GNC // 2 - 1)
            def _():
                load_start(2 * k + 2, ia, bufa, sem_a)

            drain(bufb, sem_b)
            write(2 * k + 1, bufb)
            return carry

        lax.fori_loop(0, GNC // 2, body, 0)

    return gath


def _sc_gather(tsrc, tdst, src_idx, dst_idx):
    res = _make_sc_gather()(tsrc, tdst, src_idx, dst_idx)
    return res[0], res[1]


@functools.lru_cache(maxsize=None)
def _make_sc_scatter():
    f32 = jnp.float32

    @functools.partial(
        pl.kernel, mesh=_sc_mesh(),
        out_type=[jax.ShapeDtypeStruct((4, ACC_R, W), f32)],
        scratch_types=[pltpu.VMEM((ZR, W), f32),
                       pltpu.VMEM((CH,), jnp.int32),
                       pltpu.VMEM((CH, W), f32),
                       pltpu.VMEM_SHARED((ACC_R, W), f32)],
    )
    def scat(pay_hbm, dst_hbm, out_hbm, zbuf, idxb, payb, acc):
        c = lax.axis_index("c")
        sid = lax.axis_index("s")
        zv = jnp.zeros((16,), f32)

        def zrow(i, carry):
            for j in range(W // 16):
                zbuf[i, pl.ds(j * 16, 16)] = zv
            return carry

        lax.fori_loop(0, ZR, zrow, 0)
        row0 = sid * STRIPE
        base0 = sid * (EP // 16)
        for qi in range(2):
            q = 2 * c + qi
            lo = q * RANGE
            for jj in range(STRIPE // ZR):
                pltpu.sync_copy(zbuf, acc.at[pl.ds(row0 + jj * ZR, ZR)])
            plsc.subcore_barrier()

            def body(i, carry):
                base = base0 + i * CH
                pltpu.sync_copy(dst_hbm.at[pl.ds(base, CH)], idxb)
                for j in range(CH // 16):
                    x = idxb[pl.ds(j * 16, 16)]
                    li = x - lo
                    ok = (li >= 0) & (li < RANGE)
                    idxb[pl.ds(j * 16, 16)] = jnp.where(ok, li, RANGE)
                pltpu.sync_copy(pay_hbm.at[pl.ds(base, CH)], payb)
                pltpu.sync_copy(payb, acc.at[idxb], add=True)
                return carry

            lax.fori_loop(0, EP // 16 // CH, body, 0)
            plsc.subcore_barrier()
            pltpu.sync_copy(acc.at[pl.ds(row0, STRIPE)],
                            out_hbm.at[q, pl.ds(row0, STRIPE)])

    return scat


def _sc_scatter(pay, dst_idx):
    return _make_sc_scatter()(pay, dst_idx)[0]


def _assemble(acc4):
    full = jnp.concatenate([acc4[q, :RANGE] for q in range(4)], axis=0)
    pad = jnp.zeros((NP - N, W), full.dtype)
    return jnp.concatenate([full[:N], pad], axis=0)


def _pad_rows(x, rows, value=0.0):
    pad = jnp.full((rows - x.shape[0],) + x.shape[1:], value, x.dtype)
    return jnp.concatenate([x, pad], axis=0)


def kernel(s, v, p, edge_index_local, d_local, r_local,
           edge_index_global, d_global, r_global, batch, params):
    f32 = jnp.float32
    s_p = _pad_rows(s, NP)
    vf_p = _pad_rows(v.transpose(0, 2, 1).reshape(N, V3), NP)

    def prep_edges(ei, d, r):
        src = _pad_rows(ei[0], EP)
        dst_g = _pad_rows(ei[1], EP)
        dst_s = jnp.concatenate(
            [ei[1], jnp.full((EP - E,), DUMP_IDX, jnp.int32)])
        drt = jnp.concatenate(
            [d[None, :], r.T, jnp.zeros((4, E), f32)], axis=0)
        drt = jnp.concatenate([drt, jnp.zeros((8, EP - E), f32)], axis=1)
        return src, dst_g, dst_s, drt

    srcL, dstLg, dstLs, drL = prep_edges(edge_index_local, d_local, r_local)
    srcG, dstGg, dstGs, drG = prep_edges(edge_index_global, d_global, r_global)

    def wts(i):
        lp = params[i]
        we1 = lp['We1']
        ws, wd, w1e = we1[:SDIM], we1[SDIM:2 * SDIM], we1[2 * SDIM:]
        be1 = lp['be1'].reshape(1, SDIM)
        w2p = jnp.zeros((SDIM, 128), f32).at[:, :SDIM + 2 * VDIM + 1].set(lp['We2'])
        be2p = jnp.zeros((1, 128), f32).at[0, :SDIM + 2 * VDIM + 1].set(lp['be2'])
        g = lp['g'].reshape(1, SDIM)
        b = lp['b'].reshape(1, SDIM)
        return ws, wd, w1e, be1, w2p, be2p, g, b

    ws, wd, w1e, be1, w2p, be2p, g, b = wts(0)
    sbar, vbar, tsrc, tdst = _node0(s_p, vf_p, g, b, ws, wd)
    gsrc, gdst = _sc_gather(tsrc, tdst, srcL, dstLg)
    msg = _edge(gsrc, gdst, drL, w1e, be1, w2p, be2p, True, False)
    agg = _assemble(_sc_scatter(msg, dstLs))

    for i in range(1, NUM_LAYERS):
        lp_prev = params[i - 1]
        ws, wd, w1e, be1, w2p, be2p, g, b = wts(i)
        w3 = jnp.kron(jnp.eye(3, dtype=f32), params[i]['Wv'])
        sbar, vbar, tsrc, tdst = _nodeu(
            sbar, vbar, agg,
            lp_prev['Wu1'], lp_prev['bu1'].reshape(1, SDIM),
            lp_prev['Wu2'], lp_prev['bu2'].reshape(1, SDIM + VDIM),
            g, b, ws, wd, w3)
        is_rbf = (i != NUM_LAYERS - 2)
        if is_rbf:
            src, dstg, dsts, drt = srcL, dstLg, dstLs, drL
        else:
            src, dstg, dsts, drt = srcG, dstGg, dstGs, drG
        gsrc, gdst = _sc_gather(tsrc, tdst, src, dstg)
        msg = _edge(gsrc, gdst, drt, w1e, be1, w2p, be2p, is_rbf, True)
        agg = _assemble(_sc_scatter(msg, dsts))

    s_f, v_f = _final(sbar, vbar, agg)
    s_out = s_f[:N]
    v_out = v_f[:N].reshape(N, 3, VDIM).transpose(0, 2, 1)
    return (s_out, v_out, p)


# speedup vs baseline: 14.4402x; 1.0127x over previous
import functools

import jax
import jax.numpy as jnp
import numpy as np
from jax import lax
from jax.experimental import pallas as pl
from jax.experimental.pallas import tpu as pltpu
from jax.experimental.pallas import tpu_sc as plsc

SDIM = 64
VDIM = 16
V3 = 48
RBF_DIM = 64
CUTOFF = 5.0
NUM_LAYERS = 5
N = 50000
E = 800000
W = 128

NP = 50176
EP = 819200
EB = 4096
NB = 1024
CH = 128
RANGE = 12500
ACC_R = 12544
STRIPE = ACC_R // 16
ZR = 98
DUMP_IDX = np.int32(1 << 28)

_MU = np.linspace(0.0, CUTOFF, RBF_DIM, dtype=np.float32).reshape(1, RBF_DIM)
_GAMMA = RBF_DIM / CUTOFF
_ATT_COL = SDIM + 2 * VDIM
_CNT_COL = SDIM + V3


def _ln_tc(x, g, b):
    mu = jnp.mean(x, axis=-1, keepdims=True)
    xc = x - mu
    var = jnp.mean(xc * xc, axis=-1, keepdims=True)
    return xc / jnp.sqrt(var + 1e-6) * g + b


def _vnorm_tc(vf):
    sq = vf * vf
    n2 = sq[:, :VDIM] + sq[:, VDIM:2 * VDIM] + sq[:, 2 * VDIM:]
    rms = jnp.sqrt(jnp.mean(n2, axis=-1, keepdims=True) + 1e-6)
    return vf / rms


def _tile3(x):
    return jnp.concatenate([x, x, x], axis=1)


def _dot(a, b):
    return jnp.dot(a, b, preferred_element_type=jnp.float32)


def _full_spec(shape):
    nd = len(shape)
    return pl.BlockSpec(shape, lambda i: (0,) * nd)


def _row_spec(width):
    return pl.BlockSpec((NB, width), lambda i: (i, 0))


def _zpad(x, width):
    return jnp.concatenate(
        [x, jnp.zeros((x.shape[0], width - x.shape[1]), x.dtype)], axis=1)


def _node0_body(s_ref, vf_ref, g_ref, b_ref, ws_ref, wd_ref,
                sbar_ref, vbar_ref, ts_ref, td_ref):
    sb = _ln_tc(s_ref[...], g_ref[...], b_ref[...])
    vb = _vnorm_tc(vf_ref[...])
    sbar_ref[...] = sb
    vbar_ref[...] = vb
    z = jnp.zeros((sb.shape[0], W - SDIM), jnp.float32)
    ts_ref[...] = jnp.concatenate(
        [_dot(sb, ws_ref[...]), z], axis=1)
    td_ref[...] = jnp.concatenate([_dot(sb, wd_ref[...]), z], axis=1)


def _node0(s_p, vf_p, g, b, ws, wd):
    f32 = jnp.float32
    return pl.pallas_call(
        _node0_body,
        grid=(NP // NB,),
        in_specs=[_row_spec(SDIM), _row_spec(V3),
                  _full_spec((1, SDIM)), _full_spec((1, SDIM)),
                  _full_spec((SDIM, SDIM)), _full_spec((SDIM, SDIM))],
        out_specs=[_row_spec(SDIM), _row_spec(V3), _row_spec(W), _row_spec(W)],
        out_shape=[jax.ShapeDtypeStruct((NP, SDIM), f32),
                   jax.ShapeDtypeStruct((NP, V3), f32),
                   jax.ShapeDtypeStruct((NP, W), f32),
                   jax.ShapeDtypeStruct((NP, W), f32)],
    )(s_p, vf_p, g, b, ws, wd)


def _nodeu_body(sp_ref, vp_ref, agg_ref,
                wu1_ref, bu1_ref, wu2_ref, bu2_ref,
                g_ref, b_ref, ws_ref, wd_ref, w3_ref,
                sbar_ref, vbar_ref, ts_ref, td_ref):
    agg = agg_ref[...]
    cnt = jnp.maximum(agg[:, _CNT_COL:_CNT_COL + 1], 1.0)
    s_agg = agg[:, :SDIM] / cnt
    v_agg = agg[:, SDIM:SDIM + V3] / cnt
    sq = v_agg * v_agg
    n2 = sq[:, :VDIM] + sq[:, VDIM:2 * VDIM] + sq[:, 2 * VDIM:]
    vn = jnp.sqrt(n2 + 1e-6)
    sp = sp_ref[...]
    cat = jnp.concatenate([sp, s_agg, vn], axis=1)
    u = jax.nn.silu(_dot(cat, wu1_ref[...]) + bu1_ref[...])
    u2 = _dot(u, wu2_ref[...]) + bu2_ref[...]
    s_new = sp + u2[:, :SDIM]
    gate = u2[:, SDIM:SDIM + VDIM]
    v_new = vp_ref[...] + _tile3(gate) * v_agg
    sb = _ln_tc(s_new, g_ref[...], b_ref[...])
    vb = _vnorm_tc(v_new)
    sbar_ref[...] = sb
    vbar_ref[...] = vb
    z = jnp.zeros((sb.shape[0], W - SDIM - V3), jnp.float32)
    ts_ref[...] = jnp.concatenate(
        [_dot(sb, ws_ref[...]), _dot(vb, w3_ref[...]), z], axis=1)
    z2 = jnp.zeros((sb.shape[0], W - SDIM), jnp.float32)
    td_ref[...] = jnp.concatenate([_dot(sb, wd_ref[...]), z2], axis=1)


def _nodeu(sp, vp, agg, wu1, bu1, wu2, bu2, g, b, ws, wd, w3):
    f32 = jnp.float32
    return pl.pallas_call(
        _nodeu_body,
        grid=(NP // NB,),
        in_specs=[_row_spec(SDIM), _row_spec(V3), _row_spec(W),
                  _full_spec((2 * SDIM + VDIM, SDIM)), _full_spec((1, SDIM)),
                  _full_spec((SDIM, SDIM + VDIM)), _full_spec((1, SDIM + VDIM)),
                  _full_spec((1, SDIM)), _full_spec((1, SDIM)),
                  _full_spec((SDIM, SDIM)), _full_spec((SDIM, SDIM)),
                  _full_spec((V3, V3))],
        out_specs=[_row_spec(SDIM), _row_spec(V3), _row_spec(W), _row_spec(W)],
        out_shape=[jax.ShapeDtypeStruct((NP, SDIM), f32),
                   jax.ShapeDtypeStruct((NP, V3), f32),
                   jax.ShapeDtypeStruct((NP, W), f32),
                   jax.ShapeDtypeStruct((NP, W), f32)],
    )(sp, vp, agg, wu1, bu1, wu2, bu2, g, b, ws, wd, w3)


def _fin_body(sp_ref, vp_ref, agg_ref, s_ref, v_ref):
    agg = agg_ref[...]
    cnt = jnp.maximum(agg[:, _CNT_COL:_CNT_COL + 1], 1.0)
    s_ref[...] = sp_ref[...] + agg[:, :SDIM] / cnt
    v_ref[...] = vp_ref[...] + agg[:, SDIM:SDIM + V3] / cnt


def _final(sp, vp, agg):
    f32 = jnp.float32
    return pl.pallas_call(
        _fin_body,
        grid=(NP // NB,),
        in_specs=[_row_spec(SDIM), _row_spec(V3), _row_spec(W)],
        out_specs=[_row_spec(SDIM), _row_spec(V3)],
        out_shape=[jax.ShapeDtypeStruct((NP, SDIM), f32),
                   jax.ShapeDtypeStruct((NP, V3), f32)],
    )(sp, vp, agg)


def _edge_body(gs_ref, gd_ref, drt_ref, mu_ref, w1e_ref, be1_ref, w2_ref,
               be2_ref, msg_ref, *, is_rbf, has_v):
    t = jnp.swapaxes(drt_ref[...], 0, 1)
    d = t[:, 0:1]
    gs = gs_ref[...]
    ga = gs[:, :SDIM]
    gb = gd_ref[:, :SDIM]
    if is_rbf:
        ef = jnp.exp(-_GAMMA * (d - mu_ref[...]) ** 2)
        pre = ga + gb + _dot(ef, w1e_ref[...])
    else:
        pre = ga + gb + d * w1e_ref[...]
    h = jax.nn.silu(pre + be1_ref[...])
    m = _dot(h, w2_ref[...]) + be2_ref[...]
    att = jax.nn.sigmoid(m[:, _ATT_COL:_ATT_COL + 1])
    if is_rbf:
        env = jnp.where(d < CUTOFF,
                        0.5 * (jnp.cos(jnp.pi / CUTOFF * d) + 1.0), 0.0)
        att = att * env
    sm = m[:, :SDIM] * att
    grv = m[:, SDIM + VDIM:SDIM + 2 * VDIM] * att
    nrow = t.shape[0]
    r48 = jnp.concatenate([jnp.broadcast_to(t[:, 1:2], (nrow, VDIM)),
                           jnp.broadcast_to(t[:, 2:3], (nrow, VDIM)),
                           jnp.broadcast_to(t[:, 3:4], (nrow, VDIM))], axis=1)
    vm = _tile3(grv) * r48
    if has_v:
        gvv = m[:, SDIM:SDIM + VDIM] * att
        vm = vm + _tile3(gvv) * gs[:, SDIM:SDIM + V3]
    one = jnp.ones((nrow, 1), jnp.float32)
    z = jnp.zeros((nrow, W - _CNT_COL - 1), jnp.float32)
    msg_ref[...] = jnp.concatenate([sm, vm, one, z], axis=1)


def _edge(gsrc, gdst, drt, w1e, be1, w2p, be2p, is_rbf, has_v):
    f32 = jnp.float32
    body = functools.partial(_edge_body, is_rbf=is_rbf, has_v=has_v)
    return pl.pallas_call(
        body,
        grid=(EP // EB,),
        in_specs=[pl.BlockSpec((EB, W), lambda i: (i, 0)),
                  pl.BlockSpec((EB, W), lambda i: (i, 0)),
                  pl.BlockSpec((8, EB), lambda i: (0, i)),
                  _full_spec((1, RBF_DIM)),
                  _full_spec(tuple(w1e.shape)), _full_spec((1, SDIM)),
                  _full_spec((SDIM, 128)), _full_spec((1, 128))],
        out_specs=[pl.BlockSpec((EB, W), lambda i: (i, 0))],
        out_shape=[jax.ShapeDtypeStruct((EP, W), f32)],
    )(gsrc, gdst, drt, jnp.asarray(_MU), w1e, be1, w2p, be2p)[0]


def _sc_mesh():
    return plsc.VectorSubcoreMesh(core_axis_name="c", subcore_axis_name="s")


GCH = 256
GNC = EP // 16 // GCH


@functools.lru_cache(maxsize=None)
def _make_sc_gather():
    f32 = jnp.float32

    @functools.partial(
        pl.kernel, mesh=_sc_mesh(),
        out_type=[jax.ShapeDtypeStruct((EP, W), f32),
                  jax.ShapeDtypeStruct((EP, W), f32)],
        scratch_types=[pltpu.VMEM((GCH,), jnp.int32),
                       pltpu.VMEM((GCH,), jnp.int32),
                       pltpu.VMEM((GCH, W), f32),
                       pltpu.VMEM((GCH, W), f32),
                       pltpu.SemaphoreType.DMA,
                       pltpu.SemaphoreType.DMA],
    )
    def gath(ts_hbm, td_hbm, src_hbm, dst_hbm, os_hbm, od_hbm,
             ia, ib, bufa, bufb, sem_a, sem_b):
        wid = lax.axis_index("s") * 2 + lax.axis_index("c")
        half = wid // 16
        lane = wid % 16
        base0 = lane * (EP // 16)

        def load_start(cidx, ibuf, rbuf, sem):
            base = base0 + cidx * GCH

            @pl.when(half == 0)
            def _():
                pltpu.sync_copy(src_hbm.at[pl.ds(base, GCH)], ibuf)
                pltpu.async_copy(ts_hbm.at[ibuf.at[pl.ds(0, CH)]],
                                 rbuf.at[pl.ds(0, CH)], sem)
                pltpu.async_copy(ts_hbm.at[ibuf.at[pl.ds(CH, CH)]],
                                 rbuf.at[pl.ds(CH, CH)], sem)

            @pl.when(half == 1)
            def _():
                pltpu.sync_copy(dst_hbm.at[pl.ds(base, GCH)], ibuf)
                pltpu.async_copy(td_hbm.at[ibuf.at[pl.ds(0, CH)]],
                                 rbuf.at[pl.ds(0, CH)], sem)
                pltpu.async_copy(td_hbm.at[ibuf.at[pl.ds(CH, CH)]],
                                 rbuf.at[pl.ds(CH, CH)], sem)

        def drain(rbuf, sem):
            pltpu.make_async_copy(ts_hbm.at[pl.ds(0, CH)],
                                  rbuf.at[pl.ds(0, CH)], sem).wait()
            pltpu.make_async_copy(ts_hbm.at[pl.ds(0, CH)],
                                  rbuf.at[pl.ds(CH, CH)], sem).wait()

        def write(cidx, rbuf):
            base = base0 + cidx * GCH

            @pl.when(half == 0)
            def _():
                pltpu.sync_copy(rbuf, os_hbm.at[pl.ds(base, GCH)])

            @pl.when(half == 1)
            def _():
                pltpu.sync_copy(rbuf, od_hbm.at[pl.ds(base, GCH)])

        load_start(0, ia, bufa, sem_a)

        def body(k, carry):
            load_start(2 * k + 1, ib, bufb, sem_b)
            drain(bufa, sem_a)
            write(2 * k, bufa)

            @pl.when(k < GNC // 2 - 1)
            def _():
                load_start(2 * k + 2, ia, bufa, sem_a)

            drain(bufb, sem_b)
            write(2 * k + 1, bufb)
            return carry

        lax.fori_loop(0, GNC // 2, body, 0)

    return gath


def _sc_gather(tsrc, tdst, src_idx, dst_idx):
    res = _make_sc_gather()(tsrc, tdst, src_idx, dst_idx)
    return res[0], res[1]


@functools.lru_cache(maxsize=None)
def _make_sc_scatter():
    f32 = jnp.float32

    @functools.partial(
        pl.kernel, mesh=_sc_mesh(),
        out_type=[jax.ShapeDtypeStruct((4, ACC_R, W), f32)],
        scratch_types=[pltpu.VMEM((ZR, W), f32),
                       pltpu.VMEM((CH,), jnp.int32),
                       pltpu.VMEM((CH, W), f32),
                       pltpu.VMEM_SHARED((ACC_R, W), f32)],
    )
    def scat(pay_hbm, dst_hbm, out_hbm, zbuf, idxb, payb, acc):
        c = lax.axis_index("c")
        sid = lax.axis_index("s")
        zv = jnp.zeros((16,), f32)

        def zrow(i, carry):
            for j in range(W // 16):
                zbuf[i, pl.ds(j * 16, 16)] = zv
            return carry

        lax.fori_loop(0, ZR, zrow, 0)
        row0 = sid * STRIPE
        base0 = sid * (EP // 16)
        for qi in range(2):
            q = 2 * c + qi
            lo = q * RANGE
            for jj in range(STRIPE // ZR):
                pltpu.sync_copy(zbuf, acc.at[pl.ds(row0 + jj * ZR, ZR)])
            plsc.subcore_barrier()

            def body(i, carry):
                base = base0 + i * CH
                pltpu.sync_copy(dst_hbm.at[pl.ds(base, CH)], idxb)
                for j in range(CH // 16):
                    x = idxb[pl.ds(j * 16, 16)]
                    li = x - lo
                    ok = (li >= 0) & (li < RANGE)
                    idxb[pl.ds(j * 16, 16)] = jnp.where(ok, li, RANGE)
                pltpu.sync_copy(pay_hbm.at[pl.ds(base, CH)], payb)
                pltpu.sync_copy(payb, acc.at[idxb], add=True)
                return carry

            lax.fori_loop(0, EP // 16 // CH, body, 0)
            plsc.subcore_barrier()
            pltpu.sync_copy(acc.at[pl.ds(row0, STRIPE)],
                            out_hbm.at[q, pl.ds(row0, STRIPE)])

    return scat


def _sc_scatter(pay, dst_idx):
    return _make_sc_scatter()(pay, dst_idx)[0]


def _assemble(acc4):
    full = jnp.concatenate([acc4[q, :RANGE] for q in range(4)], axis=0)
    pad = jnp.zeros((NP - N, W), full.dtype)
    return jnp.concatenate([full[:N], pad], axis=0)


def _pad_rows(x, rows, value=0.0):
    pad = jnp.full((rows - x.shape[0],) + x.shape[1:], value, x.dtype)
    return jnp.concatenate([x, pad], axis=0)


def kernel(s, v, p, edge_index_local, d_local, r_local,
           edge_index_global, d_global, r_global, batch, params):
    f32 = jnp.float32
    s_p = _pad_rows(s, NP)
    vf_p = _pad_rows(v.transpose(0, 2, 1).reshape(N, V3), NP)

    def prep_edges(ei, d, r):
        src = _pad_rows(ei[0], EP)
        dst_g = _pad_rows(ei[1], EP)
        dst_s = jnp.concatenate(
            [ei[1], jnp.full((EP - E,), DUMP_IDX, jnp.int32)])
        drt = jnp.concatenate(
            [d[None, :], r.T, jnp.zeros((4, E), f32)], axis=0)
        drt = jnp.concatenate([drt, jnp.zeros((8, EP - E), f32)], axis=1)
        return src, dst_g, dst_s, drt

    srcL, dstLg, dstLs, drL = prep_edges(edge_index_local, d_local, r_local)
    srcG, dstGg, dstGs, drG = prep_edges(edge_index_global, d_global, r_global)

    def wts(i):
        lp = params[i]
        we1 = lp['We1']
        ws, wd, w1e = we1[:SDIM], we1[SDIM:2 * SDIM], we1[2 * SDIM:]
        be1 = lp['be1'].reshape(1, SDIM)
        w2p = jnp.zeros((SDIM, 128), f32).at[:, :SDIM + 2 * VDIM + 1].set(lp['We2'])
        be2p = jnp.zeros((1, 128), f32).at[0, :SDIM + 2 * VDIM + 1].set(lp['be2'])
        g = lp['g'].reshape(1, SDIM)
        b = lp['b'].reshape(1, SDIM)
        return ws, wd, w1e, be1, w2p, be2p, g, b

    ws, wd, w1e, be1, w2p, be2p, g, b = wts(0)
    sbar, vbar, tsrc, tdst = _node0(s_p, vf_p, g, b, ws, wd)
    gsrc, gdst = _sc_gather(tsrc, tdst, srcL, dstLg)
    msg = _edge(gsrc, gdst, drL, w1e, be1, w2p, be2p, True, False)
    agg = _assemble(_sc_scatter(msg, dstLs))

    for i in range(1, NUM_LAYERS):
        lp_prev = params[i - 1]
        ws, wd, w1e, be1, w2p, be2p, g, b = wts(i)
        w3 = jnp.kron(jnp.eye(3, dtype=f32), params[i]['Wv'])
        sbar, vbar, tsrc, tdst = _nodeu(
            sbar, vbar, agg,
            lp_prev['Wu1'], lp_prev['bu1'].reshape(1, SDIM),
            lp_prev['Wu2'], lp_prev['bu2'].reshape(1, SDIM + VDIM),
            g, b, ws, wd, w3)
        is_rbf = (i != NUM_LAYERS - 2)
        if is_rbf:
            src, dstg, dsts, drt = srcL, dstLg, dstLs, drL
        else:
            src, dstg, dsts, drt = srcG, dstGg, dstGs, drG
        gsrc, gdst = _sc_gather(tsrc, tdst, src, dstg)
        msg = _edge(gsrc, gdst, drt, w1e, be1, w2p, be2p, is_rbf, True)
        agg = _assemble(_sc_scatter(msg, dsts))

    s_f, v_f = _final(sbar, vbar, agg)
    s_out = s_f[:N]
    v_out = v_f[:N].reshape(N, 3, VDIM).transpose(0, 2, 1)
    return (s_out, v_out, p)


# speedup vs baseline: 19.9525x; 1.3817x over previous
import functools

import jax
import jax.numpy as jnp
import numpy as np
from jax import lax
from jax.experimental import pallas as pl
from jax.experimental.pallas import tpu as pltpu
from jax.experimental.pallas import tpu_sc as plsc

SDIM = 64
VDIM = 16
V3 = 48
RBF_DIM = 64
CUTOFF = 5.0
NUM_LAYERS = 5
N = 50000
E = 800000
W = 128

NP = 50176
EP = 819200
EB = 4096
NB = 1024
CH = 128
RANGE = 12500
ACC_R = 12544
STRIPE = ACC_R // 16
ZR = 98
DUMP_IDX = np.int32(1 << 28)

_MU = np.linspace(0.0, CUTOFF, RBF_DIM, dtype=np.float32).reshape(1, RBF_DIM)
_GAMMA = RBF_DIM / CUTOFF
_ATT_COL = SDIM + 2 * VDIM
_CNT_COL = SDIM + V3


def _ln_tc(x, g, b):
    mu = jnp.mean(x, axis=-1, keepdims=True)
    xc = x - mu
    var = jnp.mean(xc * xc, axis=-1, keepdims=True)
    return xc / jnp.sqrt(var + 1e-6) * g + b


def _vnorm_tc(vf):
    sq = vf * vf
    n2 = sq[:, :VDIM] + sq[:, VDIM:2 * VDIM] + sq[:, 2 * VDIM:]
    rms = jnp.sqrt(jnp.mean(n2, axis=-1, keepdims=True) + 1e-6)
    return vf / rms


def _tile3(x):
    return jnp.concatenate([x, x, x], axis=1)


def _dot(a, b):
    return jnp.dot(a, b, preferred_element_type=jnp.float32)


def _full_spec(shape):
    nd = len(shape)
    return pl.BlockSpec(shape, lambda i: (0,) * nd)


def _row_spec(width):
    return pl.BlockSpec((NB, width), lambda i: (i, 0))


def _zpad(x, width):
    return jnp.concatenate(
        [x, jnp.zeros((x.shape[0], width - x.shape[1]), x.dtype)], axis=1)


def _node0_body(s_ref, vf_ref, g_ref, b_ref, ws_ref, wd_ref,
                sbar_ref, vbar_ref, ts_ref, td_ref):
    sb = _ln_tc(s_ref[...], g_ref[...], b_ref[...])
    vb = _vnorm_tc(vf_ref[...])
    sbar_ref[...] = sb
    vbar_ref[...] = vb
    z = jnp.zeros((sb.shape[0], W - SDIM), jnp.float32)
    ts_ref[...] = jnp.concatenate(
        [_dot(sb, ws_ref[...]), z], axis=1)
    td_ref[...] = jnp.concatenate([_dot(sb, wd_ref[...]), z], axis=1)


def _node0(s_p, vf_p, g, b, ws, wd):
    f32 = jnp.float32
    return pl.pallas_call(
        _node0_body,
        grid=(NP // NB,),
        in_specs=[_row_spec(SDIM), _row_spec(V3),
                  _full_spec((1, SDIM)), _full_spec((1, SDIM)),
                  _full_spec((SDIM, SDIM)), _full_spec((SDIM, SDIM))],
        out_specs=[_row_spec(SDIM), _row_spec(V3), _row_spec(W), _row_spec(W)],
        out_shape=[jax.ShapeDtypeStruct((NP, SDIM), f32),
                   jax.ShapeDtypeStruct((NP, V3), f32),
                   jax.ShapeDtypeStruct((NP, W), f32),
                   jax.ShapeDtypeStruct((NP, W), f32)],
    )(s_p, vf_p, g, b, ws, wd)


def _nodeu_body(sp_ref, vp_ref, agg_ref,
                wu1_ref, bu1_ref, wu2_ref, bu2_ref,
                g_ref, b_ref, ws_ref, wd_ref, w3_ref,
                sbar_ref, vbar_ref, ts_ref, td_ref):
    agg = agg_ref[...]
    cnt = jnp.maximum(agg[:, _CNT_COL:_CNT_COL + 1], 1.0)
    s_agg = agg[:, :SDIM] / cnt
    v_agg = agg[:, SDIM:SDIM + V3] / cnt
    sq = v_agg * v_agg
    n2 = sq[:, :VDIM] + sq[:, VDIM:2 * VDIM] + sq[:, 2 * VDIM:]
    vn = jnp.sqrt(n2 + 1e-6)
    sp = sp_ref[...]
    cat = jnp.concatenate([sp, s_agg, vn], axis=1)
    u = jax.nn.silu(_dot(cat, wu1_ref[...]) + bu1_ref[...])
    u2 = _dot(u, wu2_ref[...]) + bu2_ref[...]
    s_new = sp + u2[:, :SDIM]
    gate = u2[:, SDIM:SDIM + VDIM]
    v_new = vp_ref[...] + _tile3(gate) * v_agg
    sb = _ln_tc(s_new, g_ref[...], b_ref[...])
    vb = _vnorm_tc(v_new)
    sbar_ref[...] = sb
    vbar_ref[...] = vb
    z = jnp.zeros((sb.shape[0], W - SDIM - V3), jnp.float32)
    ts_ref[...] = jnp.concatenate(
        [_dot(sb, ws_ref[...]), _dot(vb, w3_ref[...]), z], axis=1)
    z2 = jnp.zeros((sb.shape[0], W - SDIM), jnp.float32)
    td_ref[...] = jnp.concatenate([_dot(sb, wd_ref[...]), z2], axis=1)


def _nodeu(sp, vp, agg, wu1, bu1, wu2, bu2, g, b, ws, wd, w3):
    f32 = jnp.float32
    return pl.pallas_call(
        _nodeu_body,
        grid=(NP // NB,),
        in_specs=[_row_spec(SDIM), _row_spec(V3), _row_spec(W),
                  _full_spec((2 * SDIM + VDIM, SDIM)), _full_spec((1, SDIM)),
                  _full_spec((SDIM, SDIM + VDIM)), _full_spec((1, SDIM + VDIM)),
                  _full_spec((1, SDIM)), _full_spec((1, SDIM)),
                  _full_spec((SDIM, SDIM)), _full_spec((SDIM, SDIM)),
                  _full_spec((V3, V3))],
        out_specs=[_row_spec(SDIM), _row_spec(V3), _row_spec(W), _row_spec(W)],
        out_shape=[jax.ShapeDtypeStruct((NP, SDIM), f32),
                   jax.ShapeDtypeStruct((NP, V3), f32),
                   jax.ShapeDtypeStruct((NP, W), f32),
                   jax.ShapeDtypeStruct((NP, W), f32)],
    )(sp, vp, agg, wu1, bu1, wu2, bu2, g, b, ws, wd, w3)


def _fin_body(sp_ref, vp_ref, agg_ref, s_ref, v_ref):
    agg = agg_ref[...]
    cnt = jnp.maximum(agg[:, _CNT_COL:_CNT_COL + 1], 1.0)
    s_ref[...] = sp_ref[...] + agg[:, :SDIM] / cnt
    v_ref[...] = vp_ref[...] + agg[:, SDIM:SDIM + V3] / cnt


def _final(sp, vp, agg):
    f32 = jnp.float32
    return pl.pallas_call(
        _fin_body,
        grid=(NP // NB,),
        in_specs=[_row_spec(SDIM), _row_spec(V3), _row_spec(W)],
        out_specs=[_row_spec(SDIM), _row_spec(V3)],
        out_shape=[jax.ShapeDtypeStruct((NP, SDIM), f32),
                   jax.ShapeDtypeStruct((NP, V3), f32)],
    )(sp, vp, agg)


_S48 = np.zeros((8, V3), np.float32)
for _k in range(3):
    _S48[1 + _k, _k * VDIM:(_k + 1) * VDIM] = 1.0
_T1 = np.zeros((128, V3), np.float32)
_T2 = np.zeros((128, V3), np.float32)
for _f in range(VDIM):
    for _k in range(3):
        _T1[SDIM + VDIM + _f, _k * VDIM + _f] = 1.0
        _T2[SDIM + _f, _k * VDIM + _f] = 1.0
_ONE128 = np.ones((1, 128), np.float32)


def _edge_body(gs_ref, gd_ref, drt_ref, mu_ref, w1e_ref, be1_ref, w2_ref,
               be2_ref, s48_ref, t1_ref, t2_ref, one_ref, msg_ref,
               *, is_rbf, has_v):
    t8 = drt_ref[...]
    if is_rbf:
        drow = t8[0:1, :]
        envrow = jnp.where(
            drow < CUTOFF,
            0.5 * (jnp.cos(jnp.pi / CUTOFF * drow) + 1.0), 0.0)
    else:
        envrow = jnp.ones_like(t8[0:1, :])
    t2m = jnp.concatenate([t8[:4], envrow, t8[5:]], axis=0)
    t = jnp.swapaxes(t2m, 0, 1)
    d = t[:, 0:1]
    gs = gs_ref[...]
    ga = gs[:, :SDIM]
    gb = gd_ref[:, :SDIM]
    if is_rbf:
        ef = jnp.exp(-_GAMMA * (d - mu_ref[...]) ** 2)
        pre = ga + gb + _dot(ef, w1e_ref[...])
    else:
        pre = ga + gb + _dot(d, w1e_ref[...])
    h = jax.nn.silu(pre + be1_ref[...])
    m = _dot(h, w2_ref[...]) + be2_ref[...]
    att = jax.nn.sigmoid(m[:, _ATT_COL:_ATT_COL + 1]) * t[:, 4:5]
    att128 = _dot(att, one_ref[...])
    ma = m * att128
    sm = ma[:, :SDIM]
    r48 = _dot(t, s48_ref[...])
    vm = _dot(ma, t1_ref[...]) * r48
    if has_v:
        vm = vm + _dot(ma, t2_ref[...]) * gs[:, SDIM:SDIM + V3]
    nrow = t.shape[0]
    one = jnp.ones((nrow, 1), jnp.float32)
    z = jnp.zeros((nrow, W - _CNT_COL - 1), jnp.float32)
    msg_ref[...] = jnp.concatenate([sm, vm, one, z], axis=1)


def _edge(gsrc, gdst, drt, w1e, be1, w2p, be2p, is_rbf, has_v):
    f32 = jnp.float32
    body = functools.partial(_edge_body, is_rbf=is_rbf, has_v=has_v)
    return pl.pallas_call(
        body,
        grid=(EP // EB,),
        in_specs=[pl.BlockSpec((EB, W), lambda i: (i, 0)),
                  pl.BlockSpec((EB, W), lambda i: (i, 0)),
                  pl.BlockSpec((8, EB), lambda i: (0, i)),
                  _full_spec((1, RBF_DIM)),
                  _full_spec(tuple(w1e.shape)), _full_spec((1, SDIM)),
                  _full_spec((SDIM, 128)), _full_spec((1, 128)),
                  _full_spec((8, V3)), _full_spec((128, V3)),
                  _full_spec((128, V3)), _full_spec((1, 128))],
        out_specs=[pl.BlockSpec((EB, W), lambda i: (i, 0))],
        out_shape=[jax.ShapeDtypeStruct((EP, W), f32)],
    )(gsrc, gdst, drt, jnp.asarray(_MU), w1e, be1, w2p, be2p,
      jnp.asarray(_S48), jnp.asarray(_T1), jnp.asarray(_T2),
      jnp.asarray(_ONE128))[0]


def _sc_mesh():
    return plsc.VectorSubcoreMesh(core_axis_name="c", subcore_axis_name="s")


GCH = 256
GNC = EP // 16 // GCH


@functools.lru_cache(maxsize=None)
def _make_sc_gather():
    f32 = jnp.float32

    @functools.partial(
        pl.kernel, mesh=_sc_mesh(),
        out_type=[jax.ShapeDtypeStruct((EP, W), f32),
                  jax.ShapeDtypeStruct((EP, W), f32)],
        scratch_types=[pltpu.VMEM((GCH,), jnp.int32),
                       pltpu.VMEM((GCH,), jnp.int32),
                       pltpu.VMEM((GCH, W), f32),
                       pltpu.VMEM((GCH, W), f32),
                       pltpu.SemaphoreType.DMA,
                       pltpu.SemaphoreType.DMA],
    )
    def gath(ts_hbm, td_hbm, src_hbm, dst_hbm, os_hbm, od_hbm,
             ia, ib, bufa, bufb, sem_a, sem_b):
        wid = lax.axis_index("s") * 2 + lax.axis_index("c")
        half = wid // 16
        lane = wid % 16
        base0 = lane * (EP // 16)

        def load_start(cidx, ibuf, rbuf, sem):
            base = base0 + cidx * GCH

            @pl.when(half == 0)
            def _():
                pltpu.sync_copy(src_hbm.at[pl.ds(base, GCH)], ibuf)
                pltpu.async_copy(ts_hbm.at[ibuf.at[pl.ds(0, CH)]],
                                 rbuf.at[pl.ds(0, CH)], sem)
                pltpu.async_copy(ts_hbm.at[ibuf.at[pl.ds(CH, CH)]],
                                 rbuf.at[pl.ds(CH, CH)], sem)

            @pl.when(half == 1)
            def _():
                pltpu.sync_copy(dst_hbm.at[pl.ds(base, GCH)], ibuf)
                pltpu.async_copy(td_hbm.at[ibuf.at[pl.ds(0, CH)]],
                                 rbuf.at[pl.ds(0, CH)], sem)
                pltpu.async_copy(td_hbm.at[ibuf.at[pl.ds(CH, CH)]],
                                 rbuf.at[pl.ds(CH, CH)], sem)

        def drain(rbuf, sem):
            pltpu.make_async_copy(ts_hbm.at[pl.ds(0, CH)],
                                  rbuf.at[pl.ds(0, CH)], sem).wait()
            pltpu.make_async_copy(ts_hbm.at[pl.ds(0, CH)],
                                  rbuf.at[pl.ds(CH, CH)], sem).wait()

        def write(cidx, rbuf):
            base = base0 + cidx * GCH

            @pl.when(half == 0)
            def _():
                pltpu.sync_copy(rbuf, os_hbm.at[pl.ds(base, GCH)])

            @pl.when(half == 1)
            def _():
                pltpu.sync_copy(rbuf, od_hbm.at[pl.ds(base, GCH)])

        load_start(0, ia, bufa, sem_a)

        def body(k, carry):
            load_start(2 * k + 1, ib, bufb, sem_b)
            drain(bufa, sem_a)
            write(2 * k, bufa)

            @pl.when(k < GNC // 2 - 1)
            def _():
                load_start(2 * k + 2, ia, bufa, sem_a)

            drain(bufb, sem_b)
            write(2 * k + 1, bufb)
            return carry

        lax.fori_loop(0, GNC // 2, body, 0)

    return gath


def _sc_gather(tsrc, tdst, src_idx, dst_idx):
    res = _make_sc_gather()(tsrc, tdst, src_idx, dst_idx)
    return res[0], res[1]


@functools.lru_cache(maxsize=None)
def _make_sc_scatter():
    f32 = jnp.float32

    @functools.partial(
        pl.kernel, mesh=_sc_mesh(),
        out_type=[jax.ShapeDtypeStruct((4, ACC_R, W), f32)],
        scratch_types=[pltpu.VMEM((ZR, W), f32),
                       pltpu.VMEM((CH,), jnp.int32),
                       pltpu.VMEM((CH, W), f32),
                       pltpu.VMEM_SHARED((ACC_R, W), f32)],
    )
    def scat(pay_hbm, dst_hbm, out_hbm, zbuf, idxb, payb, acc):
        c = lax.axis_index("c")
        sid = lax.axis_index("s")
        zv = jnp.zeros((16,), f32)

        def zrow(i, carry):
            for j in range(W // 16):
                zbuf[i, pl.ds(j * 16, 16)] = zv
            return carry

        lax.fori_loop(0, ZR, zrow, 0)
        row0 = sid * STRIPE
        base0 = sid * (EP // 16)
        for qi in range(2):
            q = 2 * c + qi
            lo = q * RANGE
            for jj in range(STRIPE // ZR):
                pltpu.sync_copy(zbuf, acc.at[pl.ds(row0 + jj * ZR, ZR)])
            plsc.subcore_barrier()

            def body(i, carry):
                base = base0 + i * CH
                pltpu.sync_copy(dst_hbm.at[pl.ds(base, CH)], idxb)
                for j in range(CH // 16):
                    x = idxb[pl.ds(j * 16, 16)]
                    li = x - lo
                    ok = (li >= 0) & (li < RANGE)
                    idxb[pl.ds(j * 16, 16)] = jnp.where(ok, li, RANGE)
                pltpu.sync_copy(pay_hbm.at[pl.ds(base, CH)], payb)
                pltpu.sync_copy(payb, acc.at[idxb], add=True)
                return carry

            lax.fori_loop(0, EP // 16 // CH, body, 0)
            plsc.subcore_barrier()
            pltpu.sync_copy(acc.at[pl.ds(row0, STRIPE)],
                            out_hbm.at[q, pl.ds(row0, STRIPE)])

    return scat


def _sc_scatter(pay, dst_idx):
    return _make_sc_scatter()(pay, dst_idx)[0]


def _assemble(acc4):
    full = jnp.concatenate([acc4[q, :RANGE] for q in range(4)], axis=0)
    pad = jnp.zeros((NP - N, W), full.dtype)
    return jnp.concatenate([full[:N], pad], axis=0)


def _pad_rows(x, rows, value=0.0):
    pad = jnp.full((rows - x.shape[0],) + x.shape[1:], value, x.dtype)
    return jnp.concatenate([x, pad], axis=0)


def kernel(s, v, p, edge_index_local, d_local, r_local,
           edge_index_global, d_global, r_global, batch, params):
    f32 = jnp.float32
    s_p = _pad_rows(s, NP)
    vf_p = _pad_rows(v.transpose(0, 2, 1).reshape(N, V3), NP)

    def prep_edges(ei, d, r):
        src = _pad_rows(ei[0], EP)
        dst_g = _pad_rows(ei[1], EP)
        dst_s = jnp.concatenate(
            [ei[1], jnp.full((EP - E,), DUMP_IDX, jnp.int32)])
        drt = jnp.concatenate(
            [d[None, :], r.T, jnp.zeros((4, E), f32)], axis=0)
        drt = jnp.concatenate([drt, jnp.zeros((8, EP - E), f32)], axis=1)
        return src, dst_g, dst_s, drt

    srcL, dstLg, dstLs, drL = prep_edges(edge_index_local, d_local, r_local)
    srcG, dstGg, dstGs, drG = prep_edges(edge_index_global, d_global, r_global)

    def wts(i):
        lp = params[i]
        we1 = lp['We1']
        ws, wd, w1e = we1[:SDIM], we1[SDIM:2 * SDIM], we1[2 * SDIM:]
        be1 = lp['be1'].reshape(1, SDIM)
        w2p = jnp.zeros((SDIM, 128), f32).at[:, :SDIM + 2 * VDIM + 1].set(lp['We2'])
        be2p = jnp.zeros((1, 128), f32).at[0, :SDIM + 2 * VDIM + 1].set(lp['be2'])
        g = lp['g'].reshape(1, SDIM)
        b = lp['b'].reshape(1, SDIM)
        return ws, wd, w1e, be1, w2p, be2p, g, b

    ws, wd, w1e, be1, w2p, be2p, g, b = wts(0)
    sbar, vbar, tsrc, tdst = _node0(s_p, vf_p, g, b, ws, wd)
    gsrc, gdst = _sc_gather(tsrc, tdst, srcL, dstLg)
    msg = _edge(gsrc, gdst, drL, w1e, be1, w2p, be2p, True, False)
    agg = _assemble(_sc_scatter(msg, dstLs))

    for i in range(1, NUM_LAYERS):
        lp_prev = params[i - 1]
        ws, wd, w1e, be1, w2p, be2p, g, b = wts(i)
        w3 = jnp.kron(jnp.eye(3, dtype=f32), params[i]['Wv'])
        sbar, vbar, tsrc, tdst = _nodeu(
            sbar, vbar, agg,
            lp_prev['Wu1'], lp_prev['bu1'].reshape(1, SDIM),
            lp_prev['Wu2'], lp_prev['bu2'].reshape(1, SDIM + VDIM),
            g, b, ws, wd, w3)
        is_rbf = (i != NUM_LAYERS - 2)
        if is_rbf:
            src, dstg, dsts, drt = srcL, dstLg, dstLs, drL
        else:
            src, dstg, dsts, drt = srcG, dstGg, dstGs, drG
        gsrc, gdst = _sc_gather(tsrc, tdst, src, dstg)
        msg = _edge(gsrc, gdst, drt, w1e, be1, w2p, be2p, is_rbf, True)
        agg = _assemble(_sc_scatter(msg, dsts))

    s_f, v_f = _final(sbar, vbar, agg)
    s_out = s_f[:N]
    v_out = v_f[:N].reshape(N, 3, VDIM).transpose(0, 2, 1)
    return (s_out, v_out, p)
